# Initial kernel scaffold; baseline (speedup 1.0000x reference)
import jax
import jax.numpy as jnp
from jax.experimental import pallas as pl


def kernel(a, b):
    raise NotImplementedError("write your pallas kernel here")


# speedup vs baseline: 20.5914x; 20.5914x over previous
import functools

import jax
import jax.numpy as jnp
from jax import lax
from jax.experimental import pallas as pl
from jax.experimental.pallas import tpu as pltpu
from jax.experimental.pallas import tpu_sc as plsc

F = 1024
BINS = 30
TOPK = 512
N = 32768
L = 16
NC = 2
NS = 16
NW = NC * NS
TROWS = N // NW

_MESH = plsc.VectorSubcoreMesh(core_axis_name="c", subcore_axis_name="s")


def _wid():
    return lax.axis_index("s") * NC + lax.axis_index("c")


RA = 32
NCH_A = TROWS // RA


def _mm_body(a_hbm, b_hbm, out_hbm, buf0, buf1, acc, sem0, sem1):
    wid = _wid()
    row0 = wid * TROWS
    inf = jnp.full((L,), jnp.inf, jnp.float32)
    ninf = jnp.full((L,), -jnp.inf, jnp.float32)

    def initg(g, _):
        for inp in range(2):
            acc[pl.ds(inp * 2 * F + g * L, L)] = inf
            acc[pl.ds(inp * 2 * F + F + g * L, L)] = ninf
        return 0

    lax.fori_loop(0, F // L, initg, 0)

    for inp, src in ((0, a_hbm), (1, b_hbm)):
        for b, (buf, sem) in enumerate(((buf0, sem0), (buf1, sem1))):
            pltpu.make_async_copy(
                src.at[pl.ds((row0 + b * RA) * F, RA * F)], buf, sem).start()

        def chunk(i, buf, sem):
            pltpu.make_async_copy(
                src.at[pl.ds(row0 * F, RA * F)], buf, sem).wait()

            def grp(g, _):
                col = g * L
                mnv = acc[pl.ds(inp * 2 * F + col, L)]
                mxv = acc[pl.ds(inp * 2 * F + F + col, L)]

                def rows(r, mm):
                    mn_, mx_ = mm
                    for u in range(4):
                        v = buf[pl.ds((r * 4 + u) * F + col, L)]
                        mn_ = jnp.minimum(mn_, v)
                        mx_ = jnp.maximum(mx_, v)
                    return (mn_, mx_)

                mnv, mxv = lax.fori_loop(0, RA // 4, rows, (mnv, mxv))
                acc[pl.ds(inp * 2 * F + col, L)] = mnv
                acc[pl.ds(inp * 2 * F + F + col, L)] = mxv
                return 0

            lax.fori_loop(0, F // L, grp, 0)

            @pl.when(i + 2 < NCH_A)
            def _():
                pltpu.make_async_copy(
                    src.at[pl.ds((row0 + (i + 2) * RA) * F, RA * F)],
                    buf, sem).start()

        def outer(i2, _):
            chunk(2 * i2, buf0, sem0)
            chunk(2 * i2 + 1, buf1, sem1)
            return 0

        lax.fori_loop(0, NCH_A // 2, outer, 0)

    pltpu.sync_copy(acc, out_hbm.at[pl.ds(wid * 4 * F, 4 * F)])


_mm_call = functools.partial(
    pl.kernel,
    out_type=jax.ShapeDtypeStruct((NW * 4 * F,), jnp.float32),
    mesh=_MESH,
    scratch_types=[
        pltpu.VMEM((RA * F,), jnp.float32),
        pltpu.VMEM((RA * F,), jnp.float32),
        pltpu.VMEM((4 * F,), jnp.float32),
        pltpu.SemaphoreType.DMA,
        pltpu.SemaphoreType.DMA,
    ],
)(_mm_body)


RB = 16
NCH_B = TROWS // RB
NT = F * (BINS + 1)
NH = F * BINS


def _hist_body(a_hbm, b_hbm, mn_hbm, r30_hbm, thr_hbm, out_hbm,
               thrbuf, hist, mnbuf, r30buf, buf0, buf1, sem0, sem1):
    wid = _wid()
    row0 = wid * TROWS
    iotav = lax.iota(jnp.int32, L)
    ones = jnp.full((L,), 1.0, jnp.float32)
    zeros = jnp.full((L,), 0.0, jnp.float32)

    for inp, src in ((0, a_hbm), (1, b_hbm)):
        pltpu.sync_copy(mn_hbm.at[pl.ds(inp * F, F)], mnbuf)
        pltpu.sync_copy(r30_hbm.at[pl.ds(inp * F, F)], r30buf)
        pltpu.sync_copy(thr_hbm.at[pl.ds(inp * NT, NT)], thrbuf)

        def zg(z, _):
            hist[pl.ds(z * L, L)] = zeros
            return 0

        lax.fori_loop(0, NH // L, zg, 0)

        for b, (buf, sem) in enumerate(((buf0, sem0), (buf1, sem1))):
            pltpu.make_async_copy(
                src.at[pl.ds((row0 + b * RB) * F, RB * F)], buf, sem).start()

        def chunk(i, buf, sem):
            pltpu.make_async_copy(
                src.at[pl.ds(row0 * F, RB * F)], buf, sem).wait()

            def grp(g, _):
                col = g * L
                mnv = mnbuf[pl.ds(col, L)]
                rv = r30buf[pl.ds(col, L)]
                cb31 = (iotav + col) * (BINS + 1)
                cb30 = (iotav + col) * BINS

                def rows(r, _):
                    for u in range(2):
                        v = buf[pl.ds((r * 2 + u) * F + col, L)]
                        kg = ((v - mnv) * rv).astype(jnp.int32)
                        kg = jnp.minimum(kg, jnp.int32(BINS - 1))
                        gi = cb31 + kg
                        tlo = plsc.load_gather(thrbuf, [gi])
                        thi = plsc.load_gather(thrbuf, [gi + 1])
                        adj = ((v >= thi).astype(jnp.int32)
                               - (v < tlo).astype(jnp.int32))
                        plsc.addupdate_scatter(hist, [cb30 + kg + adj], ones)
                    return 0

                lax.fori_loop(0, RB // 2, rows, 0)
                return 0

            lax.fori_loop(0, F // L, grp, 0)

            @pl.when(i + 2 < NCH_B)
            def _():
                pltpu.make_async_copy(
                    src.at[pl.ds((row0 + (i + 2) * RB) * F, RB * F)],
                    buf, sem).start()

        def outer(i2, _):
            chunk(2 * i2, buf0, sem0)
            chunk(2 * i2 + 1, buf1, sem1)
            return 0

        lax.fori_loop(0, NCH_B // 2, outer, 0)

        pltpu.sync_copy(hist, out_hbm.at[pl.ds((wid * 2 + inp) * NH, NH)])


_hist_call = functools.partial(
    pl.kernel,
    out_type=jax.ShapeDtypeStruct((NW * 2 * NH,), jnp.float32),
    mesh=_MESH,
    compiler_params=pltpu.CompilerParams(needs_layout_passes=False),
    scratch_types=[
        pltpu.VMEM((NT,), jnp.float32),
        pltpu.VMEM((NH,), jnp.float32),
        pltpu.VMEM((F,), jnp.float32),
        pltpu.VMEM((F,), jnp.float32),
        pltpu.VMEM((RB * F,), jnp.float32),
        pltpu.VMEM((RB * F,), jnp.float32),
        pltpu.SemaphoreType.DMA,
        pltpu.SemaphoreType.DMA,
    ],
)(_hist_body)


RC = 16
NCH_C = TROWS // RC


def _fuse_body(a_hbm, b_hbm, sel_hbm, out_hbm, selbuf,
               ab0, ab1, bb0, bb1, ob0, ob1,
               sa0, sa1, sb0, sb1, so0, so1):
    wid = _wid()
    row0 = wid * TROWS
    pltpu.sync_copy(sel_hbm, selbuf)

    abufs = ((ab0, sa0), (ab1, sa1))
    bbufs = ((bb0, sb0), (bb1, sb1))
    obufs = ((ob0, so0), (ob1, so1))

    for b in range(2):
        pltpu.make_async_copy(
            a_hbm.at[pl.ds((row0 + b * RC) * F, RC * F)],
            abufs[b][0], abufs[b][1]).start()
        pltpu.make_async_copy(
            b_hbm.at[pl.ds((row0 + b * RC) * F, RC * F)],
            bbufs[b][0], bbufs[b][1]).start()

    def chunk(i, ab, sa, bb, sb, ob, so, guard_out):
        if guard_out:
            @pl.when(i >= 2)
            def _():
                pltpu.make_async_copy(
                    ob, out_hbm.at[pl.ds(row0 * F, RC * F)], so).wait()
        pltpu.make_async_copy(
            a_hbm.at[pl.ds(row0 * F, RC * F)], ab, sa).wait()
        pltpu.make_async_copy(
            b_hbm.at[pl.ds(row0 * F, RC * F)], bb, sb).wait()

        def grp(g, _):
            idxv = selbuf[pl.ds(g * L, L)]

            def rows(r, _):
                v = plsc.load_gather(ab, [idxv + r * F])
                ob[pl.ds(r * F + g * L, L)] = v
                return 0

            lax.fori_loop(0, RC, rows, 0)
            return 0

        lax.fori_loop(0, TOPK // L, grp, 0)

        def grpb(g, _):
            idxv = selbuf[pl.ds(TOPK + g * L, L)]

            def rows(r, _):
                v = plsc.load_gather(bb, [idxv + r * F])
                ob[pl.ds(r * F + TOPK + g * L, L)] = v
                return 0

            lax.fori_loop(0, RC, rows, 0)
            return 0

        lax.fori_loop(0, TOPK // L, grpb, 0)

        pltpu.make_async_copy(
            ob, out_hbm.at[pl.ds((row0 + i * RC) * F, RC * F)], so).start()

        @pl.when(i + 2 < NCH_C)
        def _():
            pltpu.make_async_copy(
                a_hbm.at[pl.ds((row0 + (i + 2) * RC) * F, RC * F)],
                ab, sa).start()
            pltpu.make_async_copy(
                b_hbm.at[pl.ds((row0 + (i + 2) * RC) * F, RC * F)],
                bb, sb).start()

    def outer(i2, _):
        chunk(2 * i2, ab0, sa0, bb0, sb0, ob0, so0, True)
        chunk(2 * i2 + 1, ab1, sa1, bb1, sb1, ob1, so1, True)
        return 0

    lax.fori_loop(0, NCH_C // 2, outer, 0)

    for b in range(2):
        pltpu.make_async_copy(
            obufs[b][0], out_hbm.at[pl.ds(row0 * F, RC * F)],
            obufs[b][1]).wait()


_fuse_call = functools.partial(
    pl.kernel,
    out_type=jax.ShapeDtypeStruct((N * F,), jnp.float32),
    mesh=_MESH,
    compiler_params=pltpu.CompilerParams(needs_layout_passes=False),
    scratch_types=[
        pltpu.VMEM((2 * TOPK,), jnp.int32),
        pltpu.VMEM((RC * F,), jnp.float32),
        pltpu.VMEM((RC * F,), jnp.float32),
        pltpu.VMEM((RC * F,), jnp.float32),
        pltpu.VMEM((RC * F,), jnp.float32),
        pltpu.VMEM((RC * F,), jnp.float32),
        pltpu.VMEM((RC * F,), jnp.float32),
        pltpu.SemaphoreType.DMA,
        pltpu.SemaphoreType.DMA,
        pltpu.SemaphoreType.DMA,
        pltpu.SemaphoreType.DMA,
        pltpu.SemaphoreType.DMA,
        pltpu.SemaphoreType.DMA,
    ],
)(_fuse_body)


def _thresholds(mn, mx, rng, safe):
    kvals = jnp.arange(1, BINS, dtype=jnp.float32)
    mn3 = mn[:, :, None]
    safe3 = safe[:, :, None]

    def gfun(x):
        norm = (x - mn3) / safe3
        return jnp.floor(norm * jnp.float32(BINS))

    def to_ord(f):
        u = lax.bitcast_convert_type(f, jnp.uint32)
        return jnp.where((u >> 31) != 0, ~u, u | jnp.uint32(0x80000000))

    def from_ord(o):
        u = jnp.where(o >= jnp.uint32(0x80000000),
                      o & jnp.uint32(0x7FFFFFFF), ~o)
        return lax.bitcast_convert_type(u, jnp.float32)

    sh = mn.shape + (BINS - 1,)
    lo = jnp.broadcast_to(to_ord(mn)[:, :, None], sh)
    hi = jnp.broadcast_to(to_ord(mx)[:, :, None], sh)

    def step(_, lh):
        lo, hi = lh
        mid = lo + (hi - lo) // 2
        ok = gfun(from_ord(mid)) >= kvals
        return (jnp.where(ok, lo, mid), jnp.where(ok, mid, hi))

    lo, hi = lax.fori_loop(0, 32, step, (lo, hi))
    t = from_ord(hi)
    t = jnp.where((rng == 0)[:, :, None], jnp.inf, t)
    neg = jnp.full(mn.shape + (1,), -jnp.inf, jnp.float32)
    pos = jnp.full(mn.shape + (1,), jnp.inf, jnp.float32)
    return jnp.concatenate([neg, t, pos], axis=2).reshape(-1)


def _entropy(c, rngv):
    p = c / c.sum(axis=1, keepdims=True)
    ent = -(p * jnp.log(p + 1e-12)).sum(axis=1)
    return jnp.where(rngv == 0, 0.0, ent)


def kernel(a, b):
    af = a.reshape(-1)
    bf = b.reshape(-1)

    mmx = _mm_call(af, bf).reshape(NW, 2, 2, F)
    mn = mmx[:, :, 0, :].min(axis=0)
    mx = mmx[:, :, 1, :].max(axis=0)
    rng = mx - mn
    safe = jnp.where(rng == 0, 1.0, rng)
    r30 = jnp.float32(BINS) / safe
    thr = _thresholds(mn, mx, rng, safe)

    parts = _hist_call(af, bf, mn.reshape(-1), r30.reshape(-1), thr)
    counts = parts.reshape(NW, 2, F, BINS).sum(axis=0)

    ha = _entropy(counts[0], rng[0])
    hb = _entropy(counts[1], rng[1])
    idx_a = jnp.argsort(-ha)[:TOPK]
    idx_b = jnp.argsort(-hb)[:TOPK]
    sel = jnp.concatenate([idx_a, idx_b]).astype(jnp.int32)

    out = _fuse_call(af, bf, sel)
    return out.reshape(N, F)


# speedup vs baseline: 23.7270x; 1.1523x over previous
import functools

import jax
import jax.numpy as jnp
from jax import lax
from jax.experimental import pallas as pl
from jax.experimental.pallas import tpu as pltpu
from jax.experimental.pallas import tpu_sc as plsc

F = 1024
BINS = 30
TOPK = 512
N = 32768
L = 16
NC = 2
NS = 16
NW = NC * NS
TROWS = N // NW

_MESH = plsc.VectorSubcoreMesh(core_axis_name="c", subcore_axis_name="s")
_NOLAYOUT = pltpu.CompilerParams(needs_layout_passes=False)


def _wid():
    return lax.axis_index("s") * NC + lax.axis_index("c")


RA = 32
NCH_A = TROWS // RA


def _mm_body(a_hbm, b_hbm, out_hbm, buf0, buf1, acc, sem0, sem1):
    wid = _wid()
    row0 = wid * TROWS
    inf = jnp.full((L,), jnp.inf, jnp.float32)
    ninf = jnp.full((L,), -jnp.inf, jnp.float32)

    def initg(g, _):
        for inp in range(2):
            acc[pl.ds(inp * 2 * F + g * L, L)] = inf
            acc[pl.ds(inp * 2 * F + F + g * L, L)] = ninf
        return 0

    lax.fori_loop(0, F // L, initg, 0)

    for inp, src in ((0, a_hbm), (1, b_hbm)):
        for b, (buf, sem) in enumerate(((buf0, sem0), (buf1, sem1))):
            pltpu.make_async_copy(
                src.at[pl.ds(row0 + b * RA, RA), :], buf, sem).start()

        def chunk(i, buf, sem):
            pltpu.make_async_copy(
                src.at[pl.ds(row0, RA), :], buf, sem).wait()

            def grp(g, _):
                col = g * L
                mnv = acc[pl.ds(inp * 2 * F + col, L)]
                mxv = acc[pl.ds(inp * 2 * F + F + col, L)]

                def rows(r, mm):
                    mn_, mx_ = mm
                    for u in range(8):
                        v = buf[r * 8 + u, pl.ds(col, L)]
                        mn_ = jnp.minimum(mn_, v)
                        mx_ = jnp.maximum(mx_, v)
                    return (mn_, mx_)

                mnv, mxv = lax.fori_loop(0, RA // 8, rows, (mnv, mxv))
                acc[pl.ds(inp * 2 * F + col, L)] = mnv
                acc[pl.ds(inp * 2 * F + F + col, L)] = mxv
                return 0

            lax.fori_loop(0, F // L, grp, 0)

            @pl.when(i + 2 < NCH_A)
            def _():
                pltpu.make_async_copy(
                    src.at[pl.ds(row0 + (i + 2) * RA, RA), :],
                    buf, sem).start()

        def outer(i2, _):
            chunk(2 * i2, buf0, sem0)
            chunk(2 * i2 + 1, buf1, sem1)
            return 0

        lax.fori_loop(0, NCH_A // 2, outer, 0)

    pltpu.sync_copy(acc, out_hbm.at[pl.ds(wid * 4 * F, 4 * F)])


_mm_call = functools.partial(
    pl.kernel,
    out_type=jax.ShapeDtypeStruct((NW * 4 * F,), jnp.float32),
    mesh=_MESH,
    scratch_types=[
        pltpu.VMEM((RA, F), jnp.float32),
        pltpu.VMEM((RA, F), jnp.float32),
        pltpu.VMEM((4 * F,), jnp.float32),
        pltpu.SemaphoreType.DMA,
        pltpu.SemaphoreType.DMA,
    ],
)(_mm_body)


RB = 16
NCH_B = TROWS // RB
NT = F * (BINS + 1)
NH = F * BINS


def _hist_body(a_hbm, b_hbm, mn_hbm, r30_hbm, thr_hbm, out_hbm,
               thrbuf, hist, mnbuf, r30buf, buf0, buf1, sem0, sem1):
    wid = _wid()
    row0 = wid * TROWS
    iotav = lax.iota(jnp.int32, L)
    ones = jnp.full((L,), 1.0, jnp.float32)
    zeros = jnp.full((L,), 0.0, jnp.float32)

    for inp, src in ((0, a_hbm), (1, b_hbm)):
        pltpu.sync_copy(mn_hbm.at[pl.ds(inp * F, F)], mnbuf)
        pltpu.sync_copy(r30_hbm.at[pl.ds(inp * F, F)], r30buf)
        pltpu.sync_copy(thr_hbm.at[pl.ds(inp * NT, NT)], thrbuf)

        def zg(z, _):
            hist[pl.ds(z * L, L)] = zeros
            return 0

        lax.fori_loop(0, NH // L, zg, 0)

        for b, (buf, sem) in enumerate(((buf0, sem0), (buf1, sem1))):
            pltpu.make_async_copy(
                src.at[pl.ds(row0 + b * RB, RB), :], buf, sem).start()

        def chunk(i, buf, sem):
            pltpu.make_async_copy(
                src.at[pl.ds(row0, RB), :], buf, sem).wait()

            def grp(g, _):
                col = g * L
                mnv = mnbuf[pl.ds(col, L)]
                rv = r30buf[pl.ds(col, L)]
                cb31p1 = (iotav + col) * (BINS + 1) + 1
                cb30 = (iotav + col) * BINS

                for r in range(RB):
                    v = buf[r, pl.ds(col, L)]
                    kg = ((v - mnv) * rv).astype(jnp.int32)
                    kg = jnp.minimum(kg, jnp.int32(BINS - 1))
                    thi = plsc.load_gather(thrbuf, [cb31p1 + kg])
                    ind = (v >= thi).astype(jnp.int32)
                    plsc.addupdate_scatter(hist, [cb30 + kg + ind], ones)
                return 0

            lax.fori_loop(0, F // L, grp, 0)

            @pl.when(i + 2 < NCH_B)
            def _():
                pltpu.make_async_copy(
                    src.at[pl.ds(row0 + (i + 2) * RB, RB), :],
                    buf, sem).start()

        def outer(i2, _):
            chunk(2 * i2, buf0, sem0)
            chunk(2 * i2 + 1, buf1, sem1)
            return 0

        lax.fori_loop(0, NCH_B // 2, outer, 0)

        pltpu.sync_copy(hist, out_hbm.at[pl.ds((wid * 2 + inp) * NH, NH)])


_hist_call = functools.partial(
    pl.kernel,
    out_type=jax.ShapeDtypeStruct((NW * 2 * NH,), jnp.float32),
    mesh=_MESH,
    compiler_params=_NOLAYOUT,
    scratch_types=[
        pltpu.VMEM((NT,), jnp.float32),
        pltpu.VMEM((NH,), jnp.float32),
        pltpu.VMEM((F,), jnp.float32),
        pltpu.VMEM((F,), jnp.float32),
        pltpu.VMEM((RB, F), jnp.float32),
        pltpu.VMEM((RB, F), jnp.float32),
        pltpu.SemaphoreType.DMA,
        pltpu.SemaphoreType.DMA,
    ],
)(_hist_body)


RC = 16
NCH_C = TROWS // RC


def _fuse_body(a_hbm, b_hbm, sel_hbm, out_hbm, selbuf,
               ab0, ab1, bb0, bb1, ob0, ob1,
               sa0, sa1, sb0, sb1, so0, so1):
    wid = _wid()
    row0 = wid * TROWS
    pltpu.sync_copy(sel_hbm, selbuf)

    obufs = ((ob0, so0), (ob1, so1))

    for b, (abuf, asem) in enumerate(((ab0, sa0), (ab1, sa1))):
        pltpu.make_async_copy(
            a_hbm.at[pl.ds(row0 + b * RC, RC), :], abuf, asem).start()
    for b, (bbuf, bsem) in enumerate(((bb0, sb0), (bb1, sb1))):
        pltpu.make_async_copy(
            b_hbm.at[pl.ds(row0 + b * RC, RC), :], bbuf, bsem).start()

    def chunk(i, ab, sa, bb, sb, ob, so):
        @pl.when(i >= 2)
        def _():
            pltpu.make_async_copy(
                ob, out_hbm.at[pl.ds(row0, RC), :], so).wait()

        pltpu.make_async_copy(
            a_hbm.at[pl.ds(row0, RC), :], ab, sa).wait()
        pltpu.make_async_copy(
            b_hbm.at[pl.ds(row0, RC), :], bb, sb).wait()

        def grp(g, _):
            idxv = selbuf[pl.ds(g * L, L)]
            for r in range(RC):
                rv = jnp.full((L,), r, jnp.int32)
                ob[r, pl.ds(g * L, L)] = plsc.load_gather(ab, [rv, idxv])
            return 0

        lax.fori_loop(0, TOPK // L, grp, 0)

        def grpb(g, _):
            idxv = selbuf[pl.ds(TOPK + g * L, L)]
            for r in range(RC):
                rv = jnp.full((L,), r, jnp.int32)
                ob[r, pl.ds(TOPK + g * L, L)] = plsc.load_gather(
                    bb, [rv, idxv])
            return 0

        lax.fori_loop(0, TOPK // L, grpb, 0)

        pltpu.make_async_copy(
            ob, out_hbm.at[pl.ds(row0 + i * RC, RC), :], so).start()

        @pl.when(i + 2 < NCH_C)
        def _():
            pltpu.make_async_copy(
                a_hbm.at[pl.ds(row0 + (i + 2) * RC, RC), :], ab, sa).start()
            pltpu.make_async_copy(
                b_hbm.at[pl.ds(row0 + (i + 2) * RC, RC), :], bb, sb).start()

    def outer(i2, _):
        chunk(2 * i2, ab0, sa0, bb0, sb0, ob0, so0)
        chunk(2 * i2 + 1, ab1, sa1, bb1, sb1, ob1, so1)
        return 0

    lax.fori_loop(0, NCH_C // 2, outer, 0)

    for b in range(2):
        pltpu.make_async_copy(
            obufs[b][0], out_hbm.at[pl.ds(row0, RC), :],
            obufs[b][1]).wait()


_fuse_call = functools.partial(
    pl.kernel,
    out_type=jax.ShapeDtypeStruct((N, F), jnp.float32),
    mesh=_MESH,
    compiler_params=_NOLAYOUT,
    scratch_types=[
        pltpu.VMEM((2 * TOPK,), jnp.int32),
        pltpu.VMEM((RC, F), jnp.float32),
        pltpu.VMEM((RC, F), jnp.float32),
        pltpu.VMEM((RC, F), jnp.float32),
        pltpu.VMEM((RC, F), jnp.float32),
        pltpu.VMEM((RC, F), jnp.float32),
        pltpu.VMEM((RC, F), jnp.float32),
        pltpu.SemaphoreType.DMA,
        pltpu.SemaphoreType.DMA,
        pltpu.SemaphoreType.DMA,
        pltpu.SemaphoreType.DMA,
        pltpu.SemaphoreType.DMA,
        pltpu.SemaphoreType.DMA,
    ],
)(_fuse_body)


def _thresholds(mn, mx, rng, safe):
    kvals = jnp.arange(1, BINS, dtype=jnp.float32)
    mn3 = mn[:, :, None]
    safe3 = safe[:, :, None]

    def gfun(x):
        norm = (x - mn3) / safe3
        return jnp.floor(norm * jnp.float32(BINS))

    def to_ord(f):
        u = lax.bitcast_convert_type(f, jnp.uint32)
        return jnp.where((u >> 31) != 0, ~u, u | jnp.uint32(0x80000000))

    def from_ord(o):
        u = jnp.where(o >= jnp.uint32(0x80000000),
                      o & jnp.uint32(0x7FFFFFFF), ~o)
        return lax.bitcast_convert_type(u, jnp.float32)

    sh = mn.shape + (BINS - 1,)
    lo = jnp.broadcast_to(to_ord(mn)[:, :, None], sh)
    hi = jnp.broadcast_to(to_ord(mx)[:, :, None], sh)

    def step(_, lh):
        lo, hi = lh
        mid = lo + (hi - lo) // 2
        ok = gfun(from_ord(mid)) >= kvals
        return (jnp.where(ok, lo, mid), jnp.where(ok, mid, hi))

    lo, hi = lax.fori_loop(0, 32, step, (lo, hi))
    t = from_ord(hi)
    t = jnp.where((rng == 0)[:, :, None], jnp.inf, t)
    neg = jnp.full(mn.shape + (1,), -jnp.inf, jnp.float32)
    pos = jnp.full(mn.shape + (1,), jnp.inf, jnp.float32)
    return jnp.concatenate([neg, t, pos], axis=2).reshape(-1)


def _entropy(c, rngv):
    p = c / c.sum(axis=1, keepdims=True)
    ent = -(p * jnp.log(p + 1e-12)).sum(axis=1)
    return jnp.where(rngv == 0, 0.0, ent)


def kernel(a, b):
    mmx = _mm_call(a, b).reshape(NW, 2, 2, F)
    mn = mmx[:, :, 0, :].min(axis=0)
    mx = mmx[:, :, 1, :].max(axis=0)
    rng = mx - mn
    safe = jnp.where(rng == 0, 1.0, rng)
    r30 = (jnp.float32(BINS) / safe) * jnp.float32(1.0 - 2.0 ** -19)
    thr = _thresholds(mn, mx, rng, safe)

    parts = _hist_call(a, b, mn.reshape(-1), r30.reshape(-1), thr)
    counts = parts.reshape(NW, 2, F, BINS).sum(axis=0)

    ha = _entropy(counts[0], rng[0])
    hb = _entropy(counts[1], rng[1])
    idx_a = jnp.argsort(-ha)[:TOPK]
    idx_b = jnp.argsort(-hb)[:TOPK]
    sel = jnp.concatenate([idx_a, idx_b]).astype(jnp.int32)

    return _fuse_call(a, b, sel)


# speedup vs baseline: 67.5995x; 2.8491x over previous
import functools

import jax
import jax.numpy as jnp
from jax import lax
from jax.experimental import pallas as pl
from jax.experimental.pallas import tpu as pltpu
from jax.experimental.pallas import tpu_sc as plsc

F = 1024
BINS = 30
TOPK = 512
N = 32768
L = 16
NC = 2
NS = 16
NW = NC * NS
TROWS = N // NW

_MESH = plsc.VectorSubcoreMesh(core_axis_name="c", subcore_axis_name="s")
_NOLAYOUT = pltpu.CompilerParams(needs_layout_passes=False)


def _wid():
    return lax.axis_index("s") * NC + lax.axis_index("c")


RA = 32
NCH_A = TROWS // RA


def _mm_body(a_hbm, b_hbm, out_hbm, buf0, buf1, acc, sem0, sem1):
    wid = _wid()
    row0 = wid * TROWS
    inf = jnp.full((L,), jnp.inf, jnp.float32)
    ninf = jnp.full((L,), -jnp.inf, jnp.float32)

    def initg(g, _):
        for inp in range(2):
            acc[pl.ds(inp * 2 * F + g * L, L)] = inf
            acc[pl.ds(inp * 2 * F + F + g * L, L)] = ninf
        return 0

    lax.fori_loop(0, F // L, initg, 0)

    for inp, src in ((0, a_hbm), (1, b_hbm)):
        for b, (buf, sem) in enumerate(((buf0, sem0), (buf1, sem1))):
            pltpu.make_async_copy(
                src.at[pl.ds(row0 + b * RA, RA), :], buf, sem).start()

        def chunk(i, buf, sem):
            pltpu.make_async_copy(
                src.at[pl.ds(row0, RA), :], buf, sem).wait()

            def grp(g, _):
                col = g * L
                mnv = acc[pl.ds(inp * 2 * F + col, L)]
                mxv = acc[pl.ds(inp * 2 * F + F + col, L)]

                def rows(r, mm):
                    mn_, mx_ = mm
                    for u in range(8):
                        v = buf[r * 8 + u, pl.ds(col, L)]
                        mn_ = jnp.minimum(mn_, v)
                        mx_ = jnp.maximum(mx_, v)
                    return (mn_, mx_)

                mnv, mxv = lax.fori_loop(0, RA // 8, rows, (mnv, mxv))
                acc[pl.ds(inp * 2 * F + col, L)] = mnv
                acc[pl.ds(inp * 2 * F + F + col, L)] = mxv
                return 0

            lax.fori_loop(0, F // L, grp, 0)

            @pl.when(i + 2 < NCH_A)
            def _():
                pltpu.make_async_copy(
                    src.at[pl.ds(row0 + (i + 2) * RA, RA), :],
                    buf, sem).start()

        def outer(i2, _):
            chunk(2 * i2, buf0, sem0)
            chunk(2 * i2 + 1, buf1, sem1)
            return 0

        lax.fori_loop(0, NCH_A // 2, outer, 0)

    pltpu.sync_copy(acc, out_hbm.at[pl.ds(wid * 4 * F, 4 * F)])


_mm_call = functools.partial(
    pl.kernel,
    out_type=jax.ShapeDtypeStruct((NW * 4 * F,), jnp.float32),
    mesh=_MESH,
    scratch_types=[
        pltpu.VMEM((RA, F), jnp.float32),
        pltpu.VMEM((RA, F), jnp.float32),
        pltpu.VMEM((4 * F,), jnp.float32),
        pltpu.SemaphoreType.DMA,
        pltpu.SemaphoreType.DMA,
    ],
)(_mm_body)


RB = 16
NCH_B = TROWS // RB
NT = F * (BINS + 1)
NH = F * BINS


def _hist_body(a_hbm, b_hbm, mn_hbm, r30_hbm, thr_hbm, out_hbm,
               thrbuf, hist, mnbuf, r30buf, buf0, buf1, sem0, sem1):
    wid = _wid()
    row0 = wid * TROWS
    iotav = lax.iota(jnp.int32, L)
    ones = jnp.full((L,), 1.0, jnp.float32)
    zeros = jnp.full((L,), 0.0, jnp.float32)

    for inp, src in ((0, a_hbm), (1, b_hbm)):
        pltpu.sync_copy(mn_hbm.at[pl.ds(inp * F, F)], mnbuf)
        pltpu.sync_copy(r30_hbm.at[pl.ds(inp * F, F)], r30buf)
        pltpu.sync_copy(thr_hbm.at[pl.ds(inp * NT, NT)], thrbuf)

        def zg(z, _):
            hist[pl.ds(z * L, L)] = zeros
            return 0

        lax.fori_loop(0, NH // L, zg, 0)

        for b, (buf, sem) in enumerate(((buf0, sem0), (buf1, sem1))):
            pltpu.make_async_copy(
                src.at[pl.ds(row0 + b * RB, RB), :], buf, sem).start()

        def chunk(i, buf, sem):
            pltpu.make_async_copy(
                src.at[pl.ds(row0, RB), :], buf, sem).wait()

            def grp(g, _):
                col = g * L
                mnv = mnbuf[pl.ds(col, L)]
                rv = r30buf[pl.ds(col, L)]
                cb31p1 = (iotav + col) * (BINS + 1) + 1
                cb30 = (iotav + col) * BINS

                @plsc.parallel_loop(0, RB, step=1, unroll=8)
                def _(r):
                    v = buf[r, pl.ds(col, L)]
                    kg = ((v - mnv) * rv).astype(jnp.int32)
                    kg = jnp.minimum(kg, jnp.int32(BINS - 1))
                    thi = plsc.load_gather(thrbuf, [cb31p1 + kg])
                    ind = (v >= thi).astype(jnp.int32)
                    plsc.addupdate_scatter(hist, [cb30 + kg + ind], ones)

                return 0

            lax.fori_loop(0, F // L, grp, 0)

            @pl.when(i + 2 < NCH_B)
            def _():
                pltpu.make_async_copy(
                    src.at[pl.ds(row0 + (i + 2) * RB, RB), :],
                    buf, sem).start()

        def outer(i2, _):
            chunk(2 * i2, buf0, sem0)
            chunk(2 * i2 + 1, buf1, sem1)
            return 0

        lax.fori_loop(0, NCH_B // 2, outer, 0)

        pltpu.sync_copy(hist, out_hbm.at[pl.ds((wid * 2 + inp) * NH, NH)])


_hist_call = functools.partial(
    pl.kernel,
    out_type=jax.ShapeDtypeStruct((NW * 2 * NH,), jnp.float32),
    mesh=_MESH,
    compiler_params=_NOLAYOUT,
    scratch_types=[
        pltpu.VMEM((NT,), jnp.float32),
        pltpu.VMEM((NH,), jnp.float32),
        pltpu.VMEM((F,), jnp.float32),
        pltpu.VMEM((F,), jnp.float32),
        pltpu.VMEM((RB, F), jnp.float32),
        pltpu.VMEM((RB, F), jnp.float32),
        pltpu.SemaphoreType.DMA,
        pltpu.SemaphoreType.DMA,
    ],
)(_hist_body)


RC = 16
NCH_C = TROWS // RC


def _fuse_body(a_hbm, b_hbm, sel_hbm, out_hbm, selbuf,
               ab0, ab1, bb0, bb1, ob0, ob1,
               sa0, sa1, sb0, sb1, so0, so1):
    wid = _wid()
    row0 = wid * TROWS
    pltpu.sync_copy(sel_hbm, selbuf)

    obufs = ((ob0, so0), (ob1, so1))

    for b, (abuf, asem) in enumerate(((ab0, sa0), (ab1, sa1))):
        pltpu.make_async_copy(
            a_hbm.at[pl.ds(row0 + b * RC, RC), :], abuf, asem).start()
    for b, (bbuf, bsem) in enumerate(((bb0, sb0), (bb1, sb1))):
        pltpu.make_async_copy(
            b_hbm.at[pl.ds(row0 + b * RC, RC), :], bbuf, bsem).start()

    def chunk(i, ab, sa, bb, sb, ob, so):
        @pl.when(i >= 2)
        def _():
            pltpu.make_async_copy(
                ob, out_hbm.at[pl.ds(row0, RC), :], so).wait()

        pltpu.make_async_copy(
            a_hbm.at[pl.ds(row0, RC), :], ab, sa).wait()
        pltpu.make_async_copy(
            b_hbm.at[pl.ds(row0, RC), :], bb, sb).wait()

        def grp(g, _):
            idxv = selbuf[pl.ds(g * L, L)]

            @plsc.parallel_loop(0, RC, step=1, unroll=8)
            def _(r):
                rv = jnp.full((L,), 1, jnp.int32) * r
                ob[r, pl.ds(g * L, L)] = plsc.load_gather(ab, [rv, idxv])

            return 0

        lax.fori_loop(0, TOPK // L, grp, 0)

        def grpb(g, _):
            idxv = selbuf[pl.ds(TOPK + g * L, L)]

            @plsc.parallel_loop(0, RC, step=1, unroll=8)
            def _(r):
                rv = jnp.full((L,), 1, jnp.int32) * r
                ob[r, pl.ds(TOPK + g * L, L)] = plsc.load_gather(
                    bb, [rv, idxv])

            return 0

        lax.fori_loop(0, TOPK // L, grpb, 0)

        pltpu.make_async_copy(
            ob, out_hbm.at[pl.ds(row0 + i * RC, RC), :], so).start()

        @pl.when(i + 2 < NCH_C)
        def _():
            pltpu.make_async_copy(
                a_hbm.at[pl.ds(row0 + (i + 2) * RC, RC), :], ab, sa).start()
            pltpu.make_async_copy(
                b_hbm.at[pl.ds(row0 + (i + 2) * RC, RC), :], bb, sb).start()

    def outer(i2, _):
        chunk(2 * i2, ab0, sa0, bb0, sb0, ob0, so0)
        chunk(2 * i2 + 1, ab1, sa1, bb1, sb1, ob1, so1)
        return 0

    lax.fori_loop(0, NCH_C // 2, outer, 0)

    for b in range(2):
        pltpu.make_async_copy(
            obufs[b][0], out_hbm.at[pl.ds(row0, RC), :],
            obufs[b][1]).wait()


_fuse_call = functools.partial(
    pl.kernel,
    out_type=jax.ShapeDtypeStruct((N, F), jnp.float32),
    mesh=_MESH,
    compiler_params=_NOLAYOUT,
    scratch_types=[
        pltpu.VMEM((2 * TOPK,), jnp.int32),
        pltpu.VMEM((RC, F), jnp.float32),
        pltpu.VMEM((RC, F), jnp.float32),
        pltpu.VMEM((RC, F), jnp.float32),
        pltpu.VMEM((RC, F), jnp.float32),
        pltpu.VMEM((RC, F), jnp.float32),
        pltpu.VMEM((RC, F), jnp.float32),
        pltpu.SemaphoreType.DMA,
        pltpu.SemaphoreType.DMA,
        pltpu.SemaphoreType.DMA,
        pltpu.SemaphoreType.DMA,
        pltpu.SemaphoreType.DMA,
        pltpu.SemaphoreType.DMA,
    ],
)(_fuse_body)


def _thresholds(mn, mx, rng, safe):
    kvals = jnp.arange(1, BINS, dtype=jnp.float32)
    mn3 = mn[:, :, None]
    safe3 = safe[:, :, None]

    def gfun(x):
        norm = (x - mn3) / safe3
        return jnp.floor(norm * jnp.float32(BINS))

    def to_ord(f):
        u = lax.bitcast_convert_type(f, jnp.uint32)
        return jnp.where((u >> 31) != 0, ~u, u | jnp.uint32(0x80000000))

    def from_ord(o):
        u = jnp.where(o >= jnp.uint32(0x80000000),
                      o & jnp.uint32(0x7FFFFFFF), ~o)
        return lax.bitcast_convert_type(u, jnp.float32)

    sh = mn.shape + (BINS - 1,)
    lo = jnp.broadcast_to(to_ord(mn)[:, :, None], sh)
    hi = jnp.broadcast_to(to_ord(mx)[:, :, None], sh)

    def step(_, lh):
        lo, hi = lh
        mid = lo + (hi - lo) // 2
        ok = gfun(from_ord(mid)) >= kvals
        return (jnp.where(ok, lo, mid), jnp.where(ok, mid, hi))

    lo, hi = lax.fori_loop(0, 32, step, (lo, hi))
    t = from_ord(hi)
    t = jnp.where((rng == 0)[:, :, None], jnp.inf, t)
    neg = jnp.full(mn.shape + (1,), -jnp.inf, jnp.float32)
    pos = jnp.full(mn.shape + (1,), jnp.inf, jnp.float32)
    return jnp.concatenate([neg, t, pos], axis=2).reshape(-1)


def _entropy(c, rngv):
    p = c / c.sum(axis=1, keepdims=True)
    ent = -(p * jnp.log(p + 1e-12)).sum(axis=1)
    return jnp.where(rngv == 0, 0.0, ent)


def kernel(a, b):
    mmx = _mm_call(a, b).reshape(NW, 2, 2, F)
    mn = mmx[:, :, 0, :].min(axis=0)
    mx = mmx[:, :, 1, :].max(axis=0)
    rng = mx - mn
    safe = jnp.where(rng == 0, 1.0, rng)
    r30 = (jnp.float32(BINS) / safe) * jnp.float32(1.0 - 2.0 ** -19)
    thr = _thresholds(mn, mx, rng, safe)

    parts = _hist_call(a, b, mn.reshape(-1), r30.reshape(-1), thr)
    counts = parts.reshape(NW, 2, F, BINS).sum(axis=0)

    ha = _entropy(counts[0], rng[0])
    hb = _entropy(counts[1], rng[1])
    idx_a = jnp.argsort(-ha)[:TOPK]
    idx_b = jnp.argsort(-hb)[:TOPK]
    sel = jnp.concatenate([idx_a, idx_b]).astype(jnp.int32)

    return _fuse_call(a, b, sel)


# speedup vs baseline: 78.1709x; 1.1564x over previous
import functools

import jax
import jax.numpy as jnp
from jax import lax
from jax.experimental import pallas as pl
from jax.experimental.pallas import tpu as pltpu
from jax.experimental.pallas import tpu_sc as plsc

F = 1024
BINS = 30
TOPK = 512
N = 32768
L = 16
NC = 2
NS = 16
NW = NC * NS
TROWS = N // NW

_MESH = plsc.VectorSubcoreMesh(core_axis_name="c", subcore_axis_name="s")
_NOLAYOUT = pltpu.CompilerParams(needs_layout_passes=False)


def _wid():
    return lax.axis_index("s") * NC + lax.axis_index("c")


RA = 32
NCH_A = TROWS // RA


def _mm_body(a_hbm, b_hbm, out_hbm, buf0, buf1, acc, sem0, sem1):
    wid = _wid()
    row0 = wid * TROWS
    inf = jnp.full((L,), jnp.inf, jnp.float32)
    ninf = jnp.full((L,), -jnp.inf, jnp.float32)

    def initg(g, _):
        for inp in range(2):
            acc[pl.ds(inp * 2 * F + g * L, L)] = inf
            acc[pl.ds(inp * 2 * F + F + g * L, L)] = ninf
        return 0

    lax.fori_loop(0, F // L, initg, 0)

    for inp, src in ((0, a_hbm), (1, b_hbm)):
        for b, (buf, sem) in enumerate(((buf0, sem0), (buf1, sem1))):
            pltpu.make_async_copy(
                src.at[pl.ds(row0 + b * RA, RA), :], buf, sem).start()

        def chunk(i, buf, sem):
            pltpu.make_async_copy(
                src.at[pl.ds(row0, RA), :], buf, sem).wait()

            def grp(g, _):
                col = g * L
                mnv = acc[pl.ds(inp * 2 * F + col, L)]
                mxv = acc[pl.ds(inp * 2 * F + F + col, L)]

                def rows(r, mm):
                    mn_, mx_ = mm
                    for u in range(8):
                        v = buf[r * 8 + u, pl.ds(col, L)]
                        mn_ = jnp.minimum(mn_, v)
                        mx_ = jnp.maximum(mx_, v)
                    return (mn_, mx_)

                mnv, mxv = lax.fori_loop(0, RA // 8, rows, (mnv, mxv))
                acc[pl.ds(inp * 2 * F + col, L)] = mnv
                acc[pl.ds(inp * 2 * F + F + col, L)] = mxv
                return 0

            lax.fori_loop(0, F // L, grp, 0)

            @pl.when(i + 2 < NCH_A)
            def _():
                pltpu.make_async_copy(
                    src.at[pl.ds(row0 + (i + 2) * RA, RA), :],
                    buf, sem).start()

        def outer(i2, _):
            chunk(2 * i2, buf0, sem0)
            chunk(2 * i2 + 1, buf1, sem1)
            return 0

        lax.fori_loop(0, NCH_A // 2, outer, 0)

    pltpu.sync_copy(acc, out_hbm.at[pl.ds(wid * 4 * F, 4 * F)])


_mm_call = functools.partial(
    pl.kernel,
    out_type=jax.ShapeDtypeStruct((NW * 4 * F,), jnp.float32),
    mesh=_MESH,
    scratch_types=[
        pltpu.VMEM((RA, F), jnp.float32),
        pltpu.VMEM((RA, F), jnp.float32),
        pltpu.VMEM((4 * F,), jnp.float32),
        pltpu.SemaphoreType.DMA,
        pltpu.SemaphoreType.DMA,
    ],
)(_mm_body)


RB = 16
NCH_B = TROWS // RB
NT = F * (BINS + 1)
NH = F * BINS


def _hist_body(a_hbm, b_hbm, mn_hbm, r30_hbm, thr_hbm, out_hbm,
               thrbuf, hist, mnbuf, r30buf, buf0, buf1, sem0, sem1):
    wid = _wid()
    row0 = wid * TROWS
    iotav = lax.iota(jnp.int32, L)
    ones = jnp.full((L,), 1.0, jnp.float32)
    zeros = jnp.full((L,), 0.0, jnp.float32)

    for inp, src in ((0, a_hbm), (1, b_hbm)):
        pltpu.sync_copy(mn_hbm.at[pl.ds(inp * F, F)], mnbuf)
        pltpu.sync_copy(r30_hbm.at[pl.ds(inp * F, F)], r30buf)
        pltpu.sync_copy(thr_hbm.at[pl.ds(inp * NT, NT)], thrbuf)

        def zg(z, _):
            hist[pl.ds(z * L, L)] = zeros
            return 0

        lax.fori_loop(0, NH // L, zg, 0)

        for b, (buf, sem) in enumerate(((buf0, sem0), (buf1, sem1))):
            pltpu.make_async_copy(
                src.at[pl.ds(row0 + b * RB, RB), :], buf, sem).start()

        def chunk(i, buf, sem):
            pltpu.make_async_copy(
                src.at[pl.ds(row0, RB), :], buf, sem).wait()

            @plsc.parallel_loop(0, F // L, step=1, unroll=1)
            def grp(g):
                col = g * L
                mnv = mnbuf[pl.ds(col, L)]
                rv = r30buf[pl.ds(col, L)]
                cb31p1 = (iotav + col) * (BINS + 1) + 1
                cb30 = (iotav + col) * BINS

                @plsc.parallel_loop(0, RB, step=1, unroll=16)
                def _(r):
                    v = buf[r, pl.ds(col, L)]
                    kg = ((v - mnv) * rv).astype(jnp.int32)
                    kg = jnp.minimum(kg, jnp.int32(BINS - 1))
                    thi = plsc.load_gather(thrbuf, [cb31p1 + kg])
                    ind = (v >= thi).astype(jnp.int32)
                    plsc.addupdate_scatter(hist, [cb30 + kg + ind], ones)

            @pl.when(i + 2 < NCH_B)
            def _():
                pltpu.make_async_copy(
                    src.at[pl.ds(row0 + (i + 2) * RB, RB), :],
                    buf, sem).start()

        def outer(i2, _):
            chunk(2 * i2, buf0, sem0)
            chunk(2 * i2 + 1, buf1, sem1)
            return 0

        lax.fori_loop(0, NCH_B // 2, outer, 0)

        pltpu.sync_copy(hist, out_hbm.at[pl.ds((wid * 2 + inp) * NH, NH)])


_hist_call = functools.partial(
    pl.kernel,
    out_type=jax.ShapeDtypeStruct((NW * 2 * NH,), jnp.float32),
    mesh=_MESH,
    compiler_params=_NOLAYOUT,
    scratch_types=[
        pltpu.VMEM((NT,), jnp.float32),
        pltpu.VMEM((NH,), jnp.float32),
        pltpu.VMEM((F,), jnp.float32),
        pltpu.VMEM((F,), jnp.float32),
        pltpu.VMEM((RB, F), jnp.float32),
        pltpu.VMEM((RB, F), jnp.float32),
        pltpu.SemaphoreType.DMA,
        pltpu.SemaphoreType.DMA,
    ],
)(_hist_body)


RC = 16
NCH_C = TROWS // RC


def _fuse_body(a_hbm, b_hbm, sel_hbm, out_hbm, selbuf,
               ab0, ab1, bb0, bb1, ob0, ob1,
               sa0, sa1, sb0, sb1, so0, so1):
    wid = _wid()
    row0 = wid * TROWS
    pltpu.sync_copy(sel_hbm, selbuf)

    obufs = ((ob0, so0), (ob1, so1))

    for b, (abuf, asem) in enumerate(((ab0, sa0), (ab1, sa1))):
        pltpu.make_async_copy(
            a_hbm.at[pl.ds(row0 + b * RC, RC), :], abuf, asem).start()
    for b, (bbuf, bsem) in enumerate(((bb0, sb0), (bb1, sb1))):
        pltpu.make_async_copy(
            b_hbm.at[pl.ds(row0 + b * RC, RC), :], bbuf, bsem).start()

    def chunk(i, ab, sa, bb, sb, ob, so):
        @pl.when(i >= 2)
        def _():
            pltpu.make_async_copy(
                ob, out_hbm.at[pl.ds(row0, RC), :], so).wait()

        pltpu.make_async_copy(
            a_hbm.at[pl.ds(row0, RC), :], ab, sa).wait()
        pltpu.make_async_copy(
            b_hbm.at[pl.ds(row0, RC), :], bb, sb).wait()

        def grp(g, _):
            idxv = selbuf[pl.ds(g * L, L)]

            @plsc.parallel_loop(0, RC, step=1, unroll=8)
            def _(r):
                rv = jnp.full((L,), 1, jnp.int32) * r
                ob[r, pl.ds(g * L, L)] = plsc.load_gather(ab, [rv, idxv])

            return 0

        lax.fori_loop(0, TOPK // L, grp, 0)

        def grpb(g, _):
            idxv = selbuf[pl.ds(TOPK + g * L, L)]

            @plsc.parallel_loop(0, RC, step=1, unroll=8)
            def _(r):
                rv = jnp.full((L,), 1, jnp.int32) * r
                ob[r, pl.ds(TOPK + g * L, L)] = plsc.load_gather(
                    bb, [rv, idxv])

            return 0

        lax.fori_loop(0, TOPK // L, grpb, 0)

        pltpu.make_async_copy(
            ob, out_hbm.at[pl.ds(row0 + i * RC, RC), :], so).start()

        @pl.when(i + 2 < NCH_C)
        def _():
            pltpu.make_async_copy(
                a_hbm.at[pl.ds(row0 + (i + 2) * RC, RC), :], ab, sa).start()
            pltpu.make_async_copy(
                b_hbm.at[pl.ds(row0 + (i + 2) * RC, RC), :], bb, sb).start()

    def outer(i2, _):
        chunk(2 * i2, ab0, sa0, bb0, sb0, ob0, so0)
        chunk(2 * i2 + 1, ab1, sa1, bb1, sb1, ob1, so1)
        return 0

    lax.fori_loop(0, NCH_C // 2, outer, 0)

    for b in range(2):
        pltpu.make_async_copy(
            obufs[b][0], out_hbm.at[pl.ds(row0, RC), :],
            obufs[b][1]).wait()


_fuse_call = functools.partial(
    pl.kernel,
    out_type=jax.ShapeDtypeStruct((N, F), jnp.float32),
    mesh=_MESH,
    compiler_params=_NOLAYOUT,
    scratch_types=[
        pltpu.VMEM((2 * TOPK,), jnp.int32),
        pltpu.VMEM((RC, F), jnp.float32),
        pltpu.VMEM((RC, F), jnp.float32),
        pltpu.VMEM((RC, F), jnp.float32),
        pltpu.VMEM((RC, F), jnp.float32),
        pltpu.VMEM((RC, F), jnp.float32),
        pltpu.VMEM((RC, F), jnp.float32),
        pltpu.SemaphoreType.DMA,
        pltpu.SemaphoreType.DMA,
        pltpu.SemaphoreType.DMA,
        pltpu.SemaphoreType.DMA,
        pltpu.SemaphoreType.DMA,
        pltpu.SemaphoreType.DMA,
    ],
)(_fuse_body)


def _thresholds(mn, mx, rng, safe):
    kvals = jnp.arange(1, BINS, dtype=jnp.float32)
    mn3 = mn[:, :, None]
    safe3 = safe[:, :, None]

    def gfun(x):
        norm = (x - mn3) / safe3
        return jnp.floor(norm * jnp.float32(BINS))

    def to_ord(f):
        u = lax.bitcast_convert_type(f, jnp.uint32)
        return jnp.where((u >> 31) != 0, ~u, u | jnp.uint32(0x80000000))

    def from_ord(o):
        u = jnp.where(o >= jnp.uint32(0x80000000),
                      o & jnp.uint32(0x7FFFFFFF), ~o)
        return lax.bitcast_convert_type(u, jnp.float32)

    sh = mn.shape + (BINS - 1,)
    lo = jnp.broadcast_to(to_ord(mn)[:, :, None], sh)
    hi = jnp.broadcast_to(to_ord(mx)[:, :, None], sh)

    def step(_, lh):
        lo, hi = lh
        mid = lo + (hi - lo) // 2
        ok = gfun(from_ord(mid)) >= kvals
        return (jnp.where(ok, lo, mid), jnp.where(ok, mid, hi))

    lo, hi = lax.fori_loop(0, 32, step, (lo, hi))
    t = from_ord(hi)
    t = jnp.where((rng == 0)[:, :, None], jnp.inf, t)
    neg = jnp.full(mn.shape + (1,), -jnp.inf, jnp.float32)
    pos = jnp.full(mn.shape + (1,), jnp.inf, jnp.float32)
    return jnp.concatenate([neg, t, pos], axis=2).reshape(-1)


def _entropy(c, rngv):
    p = c / c.sum(axis=1, keepdims=True)
    ent = -(p * jnp.log(p + 1e-12)).sum(axis=1)
    return jnp.where(rngv == 0, 0.0, ent)


def kernel(a, b):
    mmx = _mm_call(a, b).reshape(NW, 2, 2, F)
    mn = mmx[:, :, 0, :].min(axis=0)
    mx = mmx[:, :, 1, :].max(axis=0)
    rng = mx - mn
    safe = jnp.where(rng == 0, 1.0, rng)
    r30 = (jnp.float32(BINS) / safe) * jnp.float32(1.0 - 2.0 ** -19)
    thr = _thresholds(mn, mx, rng, safe)

    parts = _hist_call(a, b, mn.reshape(-1), r30.reshape(-1), thr)
    counts = parts.reshape(NW, 2, F, BINS).sum(axis=0)

    ha = _entropy(counts[0], rng[0])
    hb = _entropy(counts[1], rng[1])
    idx_a = jnp.argsort(-ha)[:TOPK]
    idx_b = jnp.argsort(-hb)[:TOPK]
    sel = jnp.concatenate([idx_a, idx_b]).astype(jnp.int32)

    return _fuse_call(a, b, sel)


# speedup vs baseline: 79.7706x; 1.0205x over previous
import functools

import jax
import jax.numpy as jnp
from jax import lax
from jax.experimental import pallas as pl
from jax.experimental.pallas import tpu as pltpu
from jax.experimental.pallas import tpu_sc as plsc

F = 1024
BINS = 30
TOPK = 512
N = 32768
L = 16
NC = 2
NS = 16
NW = NC * NS
TROWS = N // NW

_MESH = plsc.VectorSubcoreMesh(core_axis_name="c", subcore_axis_name="s")
_NOLAYOUT = pltpu.CompilerParams(needs_layout_passes=False)


def _wid():
    return lax.axis_index("s") * NC + lax.axis_index("c")


RA = 32
NCH_A = TROWS // RA


def _mm_body(a_hbm, b_hbm, out_hbm, buf0, buf1, acc, sem0, sem1):
    wid = _wid()
    row0 = wid * TROWS
    inf = jnp.full((L,), jnp.inf, jnp.float32)
    ninf = jnp.full((L,), -jnp.inf, jnp.float32)

    def initg(g, _):
        for inp in range(2):
            acc[pl.ds(inp * 2 * F + g * L, L)] = inf
            acc[pl.ds(inp * 2 * F + F + g * L, L)] = ninf
        return 0

    lax.fori_loop(0, F // L, initg, 0)

    for inp, src in ((0, a_hbm), (1, b_hbm)):
        for b, (buf, sem) in enumerate(((buf0, sem0), (buf1, sem1))):
            pltpu.make_async_copy(
                src.at[pl.ds(row0 + b * RA, RA), :], buf, sem).start()

        def chunk(i, buf, sem):
            pltpu.make_async_copy(
                src.at[pl.ds(row0, RA), :], buf, sem).wait()

            def grp(g, _):
                col = g * L
                mnv = acc[pl.ds(inp * 2 * F + col, L)]
                mxv = acc[pl.ds(inp * 2 * F + F + col, L)]

                def rows(r, mm):
                    mn_, mx_ = mm
                    for u in range(8):
                        v = buf[r * 8 + u, pl.ds(col, L)]
                        mn_ = jnp.minimum(mn_, v)
                        mx_ = jnp.maximum(mx_, v)
                    return (mn_, mx_)

                mnv, mxv = lax.fori_loop(0, RA // 8, rows, (mnv, mxv))
                acc[pl.ds(inp * 2 * F + col, L)] = mnv
                acc[pl.ds(inp * 2 * F + F + col, L)] = mxv
                return 0

            lax.fori_loop(0, F // L, grp, 0)

            @pl.when(i + 2 < NCH_A)
            def _():
                pltpu.make_async_copy(
                    src.at[pl.ds(row0 + (i + 2) * RA, RA), :],
                    buf, sem).start()

        def outer(i2, _):
            chunk(2 * i2, buf0, sem0)
            chunk(2 * i2 + 1, buf1, sem1)
            return 0

        lax.fori_loop(0, NCH_A // 2, outer, 0)

    pltpu.sync_copy(acc, out_hbm.at[pl.ds(wid * 4 * F, 4 * F)])


_mm_call = functools.partial(
    pl.kernel,
    out_type=jax.ShapeDtypeStruct((NW * 4 * F,), jnp.float32),
    mesh=_MESH,
    scratch_types=[
        pltpu.VMEM((RA, F), jnp.float32),
        pltpu.VMEM((RA, F), jnp.float32),
        pltpu.VMEM((4 * F,), jnp.float32),
        pltpu.SemaphoreType.DMA,
        pltpu.SemaphoreType.DMA,
    ],
)(_mm_body)


RB = 32
NCH_B = TROWS // RB
NT = F * (BINS + 1)
NH = F * BINS


def _hist_body(a_hbm, b_hbm, mn_hbm, r30_hbm, thr_hbm, out_hbm,
               thrbuf, hist, mnbuf, r30buf, buf0, buf1, sem0, sem1):
    wid = _wid()
    row0 = wid * TROWS
    iotav = lax.iota(jnp.int32, L)
    ones = jnp.full((L,), 1.0, jnp.float32)
    zeros = jnp.full((L,), 0.0, jnp.float32)

    for inp, src in ((0, a_hbm), (1, b_hbm)):
        pltpu.sync_copy(mn_hbm.at[pl.ds(inp * F, F)], mnbuf)
        pltpu.sync_copy(r30_hbm.at[pl.ds(inp * F, F)], r30buf)
        pltpu.sync_copy(thr_hbm.at[pl.ds(inp * NT, NT)], thrbuf)

        def zg(z, _):
            hist[pl.ds(z * L, L)] = zeros
            return 0

        lax.fori_loop(0, NH // L, zg, 0)

        for b, (buf, sem) in enumerate(((buf0, sem0), (buf1, sem1))):
            pltpu.make_async_copy(
                src.at[pl.ds(row0 + b * RB, RB), :], buf, sem).start()

        def chunk(i, buf, sem):
            pltpu.make_async_copy(
                src.at[pl.ds(row0, RB), :], buf, sem).wait()

            @plsc.parallel_loop(0, F // L, step=1, unroll=2)
            def grp(g):
                col = g * L
                mnv = mnbuf[pl.ds(col, L)]
                rv = r30buf[pl.ds(col, L)]
                cb31p1 = (iotav + col) * (BINS + 1) + 1
                cb30 = (iotav + col) * BINS

                @plsc.parallel_loop(0, RB, step=1, unroll=16)
                def _(r):
                    v = buf[r, pl.ds(col, L)]
                    kg = ((v - mnv) * rv).astype(jnp.int32)
                    kg = jnp.minimum(kg, jnp.int32(BINS - 1))
                    thi = plsc.load_gather(thrbuf, [cb31p1 + kg])
                    ind = (v >= thi).astype(jnp.int32)
                    plsc.addupdate_scatter(hist, [cb30 + kg + ind], ones)

            @pl.when(i + 2 < NCH_B)
            def _():
                pltpu.make_async_copy(
                    src.at[pl.ds(row0 + (i + 2) * RB, RB), :],
                    buf, sem).start()

        def outer(i2, _):
            chunk(2 * i2, buf0, sem0)
            chunk(2 * i2 + 1, buf1, sem1)
            return 0

        lax.fori_loop(0, NCH_B // 2, outer, 0)

        pltpu.sync_copy(hist, out_hbm.at[pl.ds((wid * 2 + inp) * NH, NH)])


_hist_call = functools.partial(
    pl.kernel,
    out_type=jax.ShapeDtypeStruct((NW * 2 * NH,), jnp.float32),
    mesh=_MESH,
    compiler_params=_NOLAYOUT,
    scratch_types=[
        pltpu.VMEM((NT,), jnp.float32),
        pltpu.VMEM((NH,), jnp.float32),
        pltpu.VMEM((F,), jnp.float32),
        pltpu.VMEM((F,), jnp.float32),
        pltpu.VMEM((RB, F), jnp.float32),
        pltpu.VMEM((RB, F), jnp.float32),
        pltpu.SemaphoreType.DMA,
        pltpu.SemaphoreType.DMA,
    ],
)(_hist_body)


RC = 16
NCH_C = TROWS // RC


def _fuse_body(a_hbm, b_hbm, sel_hbm, out_hbm, selbuf,
               ab0, ab1, bb0, bb1, ob0, ob1,
               sa0, sa1, sb0, sb1, so0, so1):
    wid = _wid()
    row0 = wid * TROWS
    pltpu.sync_copy(sel_hbm, selbuf)

    obufs = ((ob0, so0), (ob1, so1))

    for b, (abuf, asem) in enumerate(((ab0, sa0), (ab1, sa1))):
        pltpu.make_async_copy(
            a_hbm.at[pl.ds(row0 + b * RC, RC), :], abuf, asem).start()
    for b, (bbuf, bsem) in enumerate(((bb0, sb0), (bb1, sb1))):
        pltpu.make_async_copy(
            b_hbm.at[pl.ds(row0 + b * RC, RC), :], bbuf, bsem).start()

    def chunk(i, ab, sa, bb, sb, ob, so):
        @pl.when(i >= 2)
        def _():
            pltpu.make_async_copy(
                ob, out_hbm.at[pl.ds(row0, RC), :], so).wait()

        pltpu.make_async_copy(
            a_hbm.at[pl.ds(row0, RC), :], ab, sa).wait()
        pltpu.make_async_copy(
            b_hbm.at[pl.ds(row0, RC), :], bb, sb).wait()

        def grp(g, _):
            idxv = selbuf[pl.ds(g * L, L)]

            @plsc.parallel_loop(0, RC, step=1, unroll=8)
            def _(r):
                rv = jnp.full((L,), 1, jnp.int32) * r
                ob[r, pl.ds(g * L, L)] = plsc.load_gather(ab, [rv, idxv])

            return 0

        lax.fori_loop(0, TOPK // L, grp, 0)

        def grpb(g, _):
            idxv = selbuf[pl.ds(TOPK + g * L, L)]

            @plsc.parallel_loop(0, RC, step=1, unroll=8)
            def _(r):
                rv = jnp.full((L,), 1, jnp.int32) * r
                ob[r, pl.ds(TOPK + g * L, L)] = plsc.load_gather(
                    bb, [rv, idxv])

            return 0

        lax.fori_loop(0, TOPK // L, grpb, 0)

        pltpu.make_async_copy(
            ob, out_hbm.at[pl.ds(row0 + i * RC, RC), :], so).start()

        @pl.when(i + 2 < NCH_C)
        def _():
            pltpu.make_async_copy(
                a_hbm.at[pl.ds(row0 + (i + 2) * RC, RC), :], ab, sa).start()
            pltpu.make_async_copy(
                b_hbm.at[pl.ds(row0 + (i + 2) * RC, RC), :], bb, sb).start()

    def outer(i2, _):
        chunk(2 * i2, ab0, sa0, bb0, sb0, ob0, so0)
        chunk(2 * i2 + 1, ab1, sa1, bb1, sb1, ob1, so1)
        return 0

    lax.fori_loop(0, NCH_C // 2, outer, 0)

    for b in range(2):
        pltpu.make_async_copy(
            obufs[b][0], out_hbm.at[pl.ds(row0, RC), :],
            obufs[b][1]).wait()


_fuse_call = functools.partial(
    pl.kernel,
    out_type=jax.ShapeDtypeStruct((N, F), jnp.float32),
    mesh=_MESH,
    compiler_params=_NOLAYOUT,
    scratch_types=[
        pltpu.VMEM((2 * TOPK,), jnp.int32),
        pltpu.VMEM((RC, F), jnp.float32),
        pltpu.VMEM((RC, F), jnp.float32),
        pltpu.VMEM((RC, F), jnp.float32),
        pltpu.VMEM((RC, F), jnp.float32),
        pltpu.VMEM((RC, F), jnp.float32),
        pltpu.VMEM((RC, F), jnp.float32),
        pltpu.SemaphoreType.DMA,
        pltpu.SemaphoreType.DMA,
        pltpu.SemaphoreType.DMA,
        pltpu.SemaphoreType.DMA,
        pltpu.SemaphoreType.DMA,
        pltpu.SemaphoreType.DMA,
    ],
)(_fuse_body)


def _thresholds(mn, mx, rng, safe):
    kvals = jnp.arange(1, BINS, dtype=jnp.float32)
    mn3 = mn[:, :, None]
    safe3 = safe[:, :, None]

    def gfun(x):
        norm = (x - mn3) / safe3
        return jnp.floor(norm * jnp.float32(BINS))

    def to_ord(f):
        u = lax.bitcast_convert_type(f, jnp.uint32)
        return jnp.where((u >> 31) != 0, ~u, u | jnp.uint32(0x80000000))

    def from_ord(o):
        u = jnp.where(o >= jnp.uint32(0x80000000),
                      o & jnp.uint32(0x7FFFFFFF), ~o)
        return lax.bitcast_convert_type(u, jnp.float32)

    sh = mn.shape + (BINS - 1,)
    lo = jnp.broadcast_to(to_ord(mn)[:, :, None], sh)
    hi = jnp.broadcast_to(to_ord(mx)[:, :, None], sh)

    def step(_, lh):
        lo, hi = lh
        mid = lo + (hi - lo) // 2
        ok = gfun(from_ord(mid)) >= kvals
        return (jnp.where(ok, lo, mid), jnp.where(ok, mid, hi))

    lo, hi = lax.fori_loop(0, 32, step, (lo, hi))
    t = from_ord(hi)
    t = jnp.where((rng == 0)[:, :, None], jnp.inf, t)
    neg = jnp.full(mn.shape + (1,), -jnp.inf, jnp.float32)
    pos = jnp.full(mn.shape + (1,), jnp.inf, jnp.float32)
    return jnp.concatenate([neg, t, pos], axis=2).reshape(-1)


def _entropy(c, rngv):
    p = c / c.sum(axis=1, keepdims=True)
    ent = -(p * jnp.log(p + 1e-12)).sum(axis=1)
    return jnp.where(rngv == 0, 0.0, ent)


def kernel(a, b):
    mmx = _mm_call(a, b).reshape(NW, 2, 2, F)
    mn = mmx[:, :, 0, :].min(axis=0)
    mx = mmx[:, :, 1, :].max(axis=0)
    rng = mx - mn
    safe = jnp.where(rng == 0, 1.0, rng)
    r30 = (jnp.float32(BINS) / safe) * jnp.float32(1.0 - 2.0 ** -19)
    thr = _thresholds(mn, mx, rng, safe)

    parts = _hist_call(a, b, mn.reshape(-1), r30.reshape(-1), thr)
    counts = parts.reshape(NW, 2, F, BINS).sum(axis=0)

    ha = _entropy(counts[0], rng[0])
    hb = _entropy(counts[1], rng[1])
    idx_a = jnp.argsort(-ha)[:TOPK]
    idx_b = jnp.argsort(-hb)[:TOPK]
    sel = jnp.concatenate([idx_a, idx_b]).astype(jnp.int32)

    return _fuse_call(a, b, sel)
